# baseline (speedup 1.0000x reference)
import jax
import jax.numpy as jnp
from jax import lax
from jax.experimental import pallas as pl
from jax.experimental.pallas import tpu as pltpu
from jax.experimental.pallas import tpu_sc as plsc

NUM_NODES = 10000
NPAD = 10240
PAD_NODE = 10200
NUM_FEAT = 128
NUM_EDGES = 320000
CHUNK = 80
NS = 16
NC = 2
NODES_PER_TILE = NPAD // NS
EDGES_PER_TILE = 20160
EPAD = NS * EDGES_PER_TILE
SEC_EDGES = 5040
N_SECS = EDGES_PER_TILE // SEC_EDGES
N_TRIPLES = SEC_EDGES // (3 * CHUNK)
NODE_BLOCK = 1024

_sc_mesh = plsc.VectorSubcoreMesh(core_axis_name="c", subcore_axis_name="s")
_sc_params = pltpu.CompilerParams(use_tc_tiling_on_sc=False)


def _deg_body(dst_hbm, o0, o1, idx_cur, ones_v, zero_v, acc):
    cid = lax.axis_index("c")
    sid = lax.axis_index("s")
    wid = cid * NS + sid
    n_chunks = dst_hbm.shape[0] // (NC * NS * CHUNK)
    ebase = wid * (dst_hbm.shape[0] // (NC * NS))
    nbase = sid * NODES_PER_TILE

    def fill_ones(i, c):
        ones_v[i] = jnp.full((16,), 1.0, jnp.float32)
        return c

    lax.fori_loop(0, CHUNK, fill_ones, 0)

    def fill_zero(i, c):
        zero_v[i] = jnp.zeros((16,), jnp.float32)
        return c

    lax.fori_loop(0, NODES_PER_TILE, fill_zero, 0)

    def zero_via(o_ref):
        pltpu.sync_copy(zero_v, o_ref.at[pl.ds(nbase, NODES_PER_TILE)])
        pltpu.sync_copy(o_ref.at[pl.ds(nbase, NODES_PER_TILE)],
                        acc.at[pl.ds(nbase, NODES_PER_TILE)])

    @pl.when(cid == 0)
    def _():
        zero_via(o0)

    @pl.when(cid == 1)
    def _():
        zero_via(o1)

    plsc.subcore_barrier()

    def body(c, carry):
        off = ebase + c * CHUNK
        pltpu.sync_copy(dst_hbm.at[pl.ds(off, CHUNK)], idx_cur)
        pltpu.sync_copy(ones_v, acc.at[idx_cur], add=True)
        return carry

    lax.fori_loop(0, n_chunks, body, 0)
    plsc.subcore_barrier()

    def writeback(o_ref):
        pltpu.sync_copy(acc.at[pl.ds(nbase, NODES_PER_TILE)],
                        o_ref.at[pl.ds(nbase, NODES_PER_TILE)])

    @pl.when(cid == 0)
    def _():
        writeback(o0)

    @pl.when(cid == 1)
    def _():
        writeback(o1)


@jax.jit
def _deg_call(dst1):
    return pl.kernel(
        _deg_body,
        out_type=(
            jax.ShapeDtypeStruct((NPAD, 16), jnp.float32),
            jax.ShapeDtypeStruct((NPAD, 16), jnp.float32),
        ),
        mesh=_sc_mesh,
        compiler_params=_sc_params,
        scratch_types=[
            pltpu.VMEM((CHUNK,), jnp.int32),
            pltpu.VMEM((CHUNK, 16), jnp.float32),
            pltpu.VMEM((NODES_PER_TILE, 16), jnp.float32),
            pltpu.VMEM_SHARED((NPAD, 16), jnp.float32),
        ],
    )(dst1)


def _prop_body(h0, h1, h2, h3, src_hbm, dst_hbm, o0, o1, o2, o3,
               src_v, dst_v, rows_a, rows_b, rows_c, acc, gsem, ssem):
    cid = lax.axis_index("c")
    sid = lax.axis_index("s")
    nbase = sid * NODES_PER_TILE
    ebase = sid * EDGES_PER_TILE

    def do_hour(h_ref, o_ref):
        pltpu.sync_copy(h_ref.at[pl.ds(nbase, NODES_PER_TILE)],
                        acc.at[pl.ds(nbase, NODES_PER_TILE)])
        plsc.subcore_barrier()

        def gather(off, rows):
            pltpu.async_copy(h_ref.at[src_v.at[pl.ds(off, CHUNK)]],
                             rows, gsem)

        def scatter(off, rows):
            pltpu.async_copy(rows, acc.at[dst_v.at[pl.ds(off, CHUNK)]],
                             ssem, add=True)

        def wait_g():
            pltpu.make_async_copy(h_ref.at[pl.ds(0, CHUNK)], rows_a,
                                  gsem).wait()

        def wait_s():
            pltpu.make_async_copy(h_ref.at[pl.ds(0, CHUNK)], rows_a,
                                  ssem).wait()

        def section(s, carry):
            sbase = pl.multiple_of(ebase + s * SEC_EDGES, 8)
            pltpu.sync_copy(src_hbm.at[pl.ds(sbase, SEC_EDGES)], src_v)
            pltpu.sync_copy(dst_hbm.at[pl.ds(sbase, SEC_EDGES)], dst_v)
            gather(0, rows_a)
            gather(CHUNK, rows_b)

            def body(q, carry):
                off0 = q * (3 * CHUNK)
                off1 = off0 + CHUNK
                off2 = off1 + CHUNK
                more = q + 1 < N_TRIPLES

                wait_g()

                @pl.when(q > 0)
                def _():
                    wait_s()

                gather(off2, rows_c)
                scatter(off0, rows_a)
                wait_g()
                wait_s()

                @pl.when(more)
                def _():
                    gather(off2 + CHUNK, rows_a)

                scatter(off1, rows_b)
                wait_g()
                wait_s()

                @pl.when(more)
                def _():
                    gather(off2 + 2 * CHUNK, rows_b)

                scatter(off2, rows_c)
                return carry

            lax.fori_loop(0, N_TRIPLES, body, 0)
            wait_s()
            return carry

        lax.fori_loop(0, N_SECS, section, 0)
        plsc.subcore_barrier()
        pltpu.sync_copy(acc.at[pl.ds(nbase, NODES_PER_TILE)],
                        o_ref.at[pl.ds(nbase, NODES_PER_TILE)])
        plsc.subcore_barrier()

    @pl.when(cid == 0)
    def _():
        do_hour(h0, o0)
        do_hour(h1, o1)

    @pl.when(cid == 1)
    def _():
        do_hour(h2, o2)
        do_hour(h3, o3)


@jax.jit
def _prop_call(h4, src1, dst1):
    node_t = jax.ShapeDtypeStruct((NPAD, NUM_FEAT), jnp.float32)
    outs = pl.kernel(
        _prop_body,
        out_type=(node_t,) * 4,
        mesh=_sc_mesh,
        compiler_params=_sc_params,
        scratch_types=[
            pltpu.VMEM((SEC_EDGES,), jnp.int32),
            pltpu.VMEM((SEC_EDGES,), jnp.int32),
            pltpu.VMEM((CHUNK, NUM_FEAT), jnp.float32),
            pltpu.VMEM((CHUNK, NUM_FEAT), jnp.float32),
            pltpu.VMEM((CHUNK, NUM_FEAT), jnp.float32),
            pltpu.VMEM_SHARED((NPAD, NUM_FEAT), jnp.float32),
            pltpu.SemaphoreType.DMA,
            pltpu.SemaphoreType.DMA,
        ],
    )(h4[0], h4[1], h4[2], h4[3], src1, dst1)
    return jnp.stack(outs, axis=0)


def _dinv_block(d0_ref, d1_ref):
    deg = d0_ref[:, :1] + d1_ref[:, :1] + 1.0
    return lax.rsqrt(deg)


def _mm1_kernel(x_ref, mask_ref, w1b_ref, d0_ref, d1_ref, w_ref, o_ref):
    dinv = _dinv_block(d0_ref, d1_ref)
    h = jnp.dot(x_ref[0], w_ref[...], preferred_element_type=jnp.float32)
    m = mask_ref[pl.ds(pl.program_id(0), 1), :]
    o_ref[0] = (h + m * w1b_ref[...]) * dinv


@jax.jit
def _mm1_call(xh, mask41, w1b, d0, d1, w1a):
    grid = (4, NPAD // NODE_BLOCK)
    return pl.pallas_call(
        _mm1_kernel,
        grid=grid,
        in_specs=[
            pl.BlockSpec((1, NODE_BLOCK, NUM_FEAT), lambda h, n: (h, n, 0)),
            pl.BlockSpec((4, 1), lambda h, n: (0, 0)),
            pl.BlockSpec((1, NUM_FEAT), lambda h, n: (0, 0)),
            pl.BlockSpec((NODE_BLOCK, 16), lambda h, n: (n, 0)),
            pl.BlockSpec((NODE_BLOCK, 16), lambda h, n: (n, 0)),
            pl.BlockSpec((NUM_FEAT, NUM_FEAT), lambda h, n: (0, 0)),
        ],
        out_specs=pl.BlockSpec((1, NODE_BLOCK, NUM_FEAT), lambda h, n: (h, n, 0)),
        out_shape=jax.ShapeDtypeStruct((4, NPAD, NUM_FEAT), jnp.float32),
    )(xh, mask41, w1b, d0, d1, w1a)


def _mm2_kernel(s_ref, b1_ref, d0_ref, d1_ref, w_ref, o_ref):
    dinv = _dinv_block(d0_ref, d1_ref)
    enc = s_ref[0] * dinv + b1_ref[...]
    o_ref[0] = jnp.dot(enc, w_ref[...], preferred_element_type=jnp.float32) * dinv


@jax.jit
def _mm2_call(s1, b1r, d0, d1, w2):
    grid = (4, NPAD // NODE_BLOCK)
    return pl.pallas_call(
        _mm2_kernel,
        grid=grid,
        in_specs=[
            pl.BlockSpec((1, NODE_BLOCK, NUM_FEAT), lambda h, n: (h, n, 0)),
            pl.BlockSpec((1, NUM_FEAT), lambda h, n: (0, 0)),
            pl.BlockSpec((NODE_BLOCK, 16), lambda h, n: (n, 0)),
            pl.BlockSpec((NODE_BLOCK, 16), lambda h, n: (n, 0)),
            pl.BlockSpec((NUM_FEAT, NUM_FEAT), lambda h, n: (0, 0)),
        ],
        out_specs=pl.BlockSpec((1, NODE_BLOCK, NUM_FEAT), lambda h, n: (h, n, 0)),
        out_shape=jax.ShapeDtypeStruct((4, NPAD, NUM_FEAT), jnp.float32),
    )(s1, b1r, d0, d1, w2)


def _fin_kernel(s_ref, b2_ref, d0_ref, d1_ref, o_ref):
    dinv = _dinv_block(d0_ref, d1_ref)
    o_ref[0] = jnp.tanh(s_ref[0] * dinv + b2_ref[...])


@jax.jit
def _fin_call(s2, b2r, d0, d1):
    grid = (4, NPAD // NODE_BLOCK)
    return pl.pallas_call(
        _fin_kernel,
        grid=grid,
        in_specs=[
            pl.BlockSpec((1, NODE_BLOCK, NUM_FEAT), lambda h, n: (h, n, 0)),
            pl.BlockSpec((1, NUM_FEAT), lambda h, n: (0, 0)),
            pl.BlockSpec((NODE_BLOCK, 16), lambda h, n: (n, 0)),
            pl.BlockSpec((NODE_BLOCK, 16), lambda h, n: (n, 0)),
        ],
        out_specs=pl.BlockSpec((1, NODE_BLOCK, NUM_FEAT), lambda h, n: (h, n, 0)),
        out_shape=jax.ShapeDtypeStruct((4, NPAD, NUM_FEAT), jnp.float32),
    )(s2, b2r, d0, d1)


def kernel(x, mask, edge_index, W1, b1, W2, b2):
    B, H, N, F = x.shape
    xh = jnp.pad(x[0], ((0, 0), (0, NPAD - N), (0, 0)))
    pad = jnp.full((EPAD - NUM_EDGES,), PAD_NODE, jnp.int32)
    src1 = jnp.concatenate([edge_index[0].astype(jnp.int32), pad])
    dst1 = jnp.concatenate([edge_index[1].astype(jnp.int32), pad])

    d0, d1 = _deg_call(dst1)

    w1a = W1[:F]
    w1b = W1[F].reshape(1, F)
    mask41 = mask[0].reshape(H, 1)

    h1 = _mm1_call(xh, mask41, w1b, d0, d1, w1a)
    s1 = _prop_call(h1, src1, dst1)
    h2 = _mm2_call(s1, b1.reshape(1, F), d0, d1, W2)
    s2 = _prop_call(h2, src1, dst1)
    out = _fin_call(s2, b2.reshape(1, F), d0, d1)
    return out[:, :N][None]
